# baseline (speedup 1.0000x reference)
import functools

import jax
import jax.numpy as jnp
from jax import lax
from jax.experimental import pallas as pl
from jax.experimental.pallas import tpu as pltpu
from jax.experimental.pallas import tpu_sc as plsc

N_NODES_ = 10000
D_ = 128
K_ = 64
E_PER = 320000
E_TOT = 2 * E_PER
NC_, NS_, LANES_ = 2, 16, 16
NW_ = NC_ * NS_
CHUNK_ = 128
CPW_ = 160
E_PAD = NW_ * CPW_ * CHUNK_


def _mm_body(x_ref, w_ref, o_ref):
    o_ref[...] = jnp.dot(x_ref[...], w_ref[...],
                         preferred_element_type=jnp.float32)


def _encode(data, W):
    return pl.pallas_call(
        _mm_body,
        out_shape=jax.ShapeDtypeStruct((N_NODES_, K_), jnp.float32),
        grid=(5,),
        in_specs=[
            pl.BlockSpec((N_NODES_ // 5, D_), lambda i: (i, 0)),
            pl.BlockSpec((D_, K_), lambda i: (0, 0)),
        ],
        out_specs=pl.BlockSpec((N_NODES_ // 5, K_), lambda i: (i, 0)),
    )(data, W)


def _sc_scores(z, srcs, dsts):
    mesh = plsc.VectorSubcoreMesh(core_axis_name="c", subcore_axis_name="s")

    @functools.partial(
        pl.kernel,
        mesh=mesh,
        compiler_params=pltpu.CompilerParams(
            needs_layout_passes=False, use_tc_tiling_on_sc=False),
        out_type=jax.ShapeDtypeStruct((E_PAD,), jnp.float32),
        scratch_types=[
            pltpu.VMEM((2, CHUNK_), jnp.int32),
            pltpu.VMEM((2, CHUNK_), jnp.int32),
            pltpu.VMEM((2, CHUNK_, K_), jnp.float32),
            pltpu.VMEM((2, CHUNK_, K_), jnp.float32),
            pltpu.VMEM((2, CHUNK_), jnp.float32),
            pltpu.SemaphoreType.DMA,
            pltpu.SemaphoreType.DMA,
        ],
    )
    def k(z_hbm, src_hbm, dst_hbm, out_hbm,
          idx_s, idx_d, rows_s, rows_d, score_v, sem0, sem1):
        wid = lax.axis_index("s") * NC_ + lax.axis_index("c")
        sems = (sem0, sem1)

        def start(c, b):
            off = (wid * CPW_ + c) * CHUNK_
            pltpu.sync_copy(src_hbm.at[pl.ds(off, CHUNK_)], idx_s.at[b])
            pltpu.sync_copy(dst_hbm.at[pl.ds(off, CHUNK_)], idx_d.at[b])
            pltpu.async_copy(z_hbm.at[idx_s.at[b]], rows_s.at[b], sems[b])
            pltpu.async_copy(z_hbm.at[idx_d.at[b]], rows_d.at[b], sems[b])

        def drain(b):
            pltpu.make_async_copy(z_hbm.at[idx_s.at[b]],
                                  rows_s.at[b], sems[b]).wait()
            pltpu.make_async_copy(z_hbm.at[idx_d.at[b]],
                                  rows_d.at[b], sems[b]).wait()

        def compute(c, b):
            def group(g, carry2):
                base = g * LANES_
                lane = lax.iota(jnp.int32, LANES_)
                res = jnp.zeros((LANES_,), jnp.float32)
                for j in range(LANES_):
                    e = base + j
                    acc = (rows_s[b, e, pl.ds(0, LANES_)]
                           * rows_d[b, e, pl.ds(0, LANES_)])
                    for q in range(1, K_ // LANES_):
                        acc = acc + (rows_s[b, e, pl.ds(q * LANES_, LANES_)]
                                     * rows_d[b, e, pl.ds(q * LANES_, LANES_)])
                    s = jnp.sum(acc)
                    res = jnp.where(lane == j, s, res)
                score_v[b, pl.ds(base, LANES_)] = res
                return carry2

            lax.fori_loop(0, CHUNK_ // LANES_, group, 0)
            off = (wid * CPW_ + c) * CHUNK_
            pltpu.sync_copy(score_v.at[b], out_hbm.at[pl.ds(off, CHUNK_)])

        start(0, 0)

        def outer(p, carry):
            c0 = p * 2
            start(c0 + 1, 1)
            drain(0)
            compute(c0, 0)

            @pl.when(p < CPW_ // 2 - 1)
            def _():
                start(c0 + 2, 0)

            drain(1)
            compute(c0 + 1, 1)
            return carry

        lax.fori_loop(0, CPW_ // 2, outer, 0)

    return k(z, srcs, dsts)


def _bce_body(x_ref, o_ref):
    x = x_ref[...]
    rows = lax.broadcasted_iota(jnp.int32, x.shape, 0)
    t = (rows < (E_PER // x.shape[1])).astype(jnp.float32)
    valid = (rows < (E_TOT // x.shape[1])).astype(jnp.float32)
    term = jnp.maximum(x, 0.0) - x * t + jnp.log1p(jnp.exp(-jnp.abs(x)))
    o_ref[...] = (jnp.sum(term * valid) * (1.0 / E_TOT)).reshape(1, 1)


def _bce_reduce(scores2d):
    return pl.pallas_call(
        _bce_body,
        out_shape=jax.ShapeDtypeStruct((1, 1), jnp.float32),
    )(scores2d)


def kernel(data, W, edges_pos, edges_neg):
    z = _encode(data, W)
    pad = jnp.zeros((E_PAD - E_TOT,), jnp.int32)
    srcs = jnp.concatenate(
        (edges_pos[0].astype(jnp.int32), edges_neg[0].astype(jnp.int32), pad))
    dsts = jnp.concatenate(
        (edges_pos[1].astype(jnp.int32), edges_neg[1].astype(jnp.int32), pad))
    scores = _sc_scores(z, srcs, dsts)
    cost = _bce_reduce(scores.reshape(E_PAD // D_, D_))
    return cost.reshape(())
